# baseline (speedup 1.0000x reference)
import functools

import jax
import jax.numpy as jnp
from jax import lax
from jax.experimental import pallas as pl
from jax.experimental.pallas import tpu as pltpu
from jax.experimental.pallas import tpu_sc as plsc

HIDDEN = 64
NUM_CORES = 2
NUM_SUBCORES = 16
NW = NUM_CORES * NUM_SUBCORES
NB = 8

_mesh = plsc.VectorSubcoreMesh(core_axis_name="c", subcore_axis_name="s")


@functools.lru_cache(maxsize=None)
def _make_gather(batch, hist):
    assert batch % (NW * NB * 2) == 0 and hist <= 128
    hp = ((hist + 7) // 8) * 8
    per_w = batch // NW
    groups = per_w // NB
    gr = NB * hp

    @functools.partial(
        pl.kernel,
        out_type=jax.ShapeDtypeStruct((batch * hp, 2 * HIDDEN), jnp.float32),
        mesh=_mesh,
        scratch_types=[
            pltpu.VMEM((2, NB, hist), jnp.int32),
            pltpu.VMEM((2, gr, 2 * HIDDEN), jnp.float32),
            pltpu.SemaphoreType.DMA,
            pltpu.SemaphoreType.DMA,
            pltpu.SemaphoreType.DMA,
            pltpu.SemaphoreType.DMA,
        ],
        compiler_params=pltpu.CompilerParams(use_tc_tiling_on_sc=False),
    )
    def gather_kernel(table_hbm, idx_hbm, out_hbm, idx_v, rows_v,
                      gsem0, gsem1, osem0, osem1):
        wid = lax.axis_index("s") * NUM_CORES + lax.axis_index("c")
        base_b = wid * per_w
        base_r = base_b * hp
        gsems = (gsem0, gsem1)
        osems = (osem0, osem1)

        def fire(b, g):
            pltpu.sync_copy(idx_hbm.at[pl.ds(base_b + g * NB, NB)], idx_v.at[b])
            for j in range(NB):
                pltpu.async_copy(table_hbm.at[idx_v.at[b].at[j]],
                                 rows_v.at[b].at[pl.ds(j * hp, hist)],
                                 gsems[b])

        def drain_gathers(b):
            for j in range(NB):
                pltpu.make_async_copy(table_hbm.at[idx_v.at[b].at[j]],
                                      rows_v.at[b].at[pl.ds(j * hp, hist)],
                                      gsems[b]).wait()

        def out_slice(g):
            return out_hbm.at[pl.ds(base_r + g * gr, gr)]

        def fire_out(b, g):
            pltpu.async_copy(rows_v.at[b], out_slice(g), osems[b])

        def wait_out(b, g):
            pltpu.make_async_copy(rows_v.at[b], out_slice(g), osems[b]).wait()

        fire(0, 0)

        @pl.loop(0, groups, step=2)
        def _grp(g0):
            for b in (0, 1):
                gg = g0 + b
                nb = 1 - b
                gf = gg + 1

                @pl.when(gf < groups)
                def _():
                    @pl.when(gg >= 1)
                    def _():
                        wait_out(nb, gg - 1)
                    fire(nb, gf)

                drain_gathers(b)
                fire_out(b, gg)

        wait_out(0, groups - 2)
        wait_out(1, groups - 1)

    return gather_kernel


def kernel(inputs, weight):
    batch, hist = inputs.shape
    hp = ((hist + 7) // 8) * 8
    idx = inputs.astype(jnp.int32)
    w128 = jnp.pad(weight, ((0, 0), (0, 2 * HIDDEN - weight.shape[1])))
    out = _make_gather(batch, hist)(w128, idx)
    return out.reshape(batch, hp, 2 * HIDDEN)[:, :hist, :HIDDEN]


# speedup vs baseline: 1.0567x; 1.0567x over previous
import functools

import jax
import jax.numpy as jnp
from jax import lax
from jax.experimental import pallas as pl
from jax.experimental.pallas import tpu as pltpu
from jax.experimental.pallas import tpu_sc as plsc

HIDDEN = 64
NUM_CORES = 2
NUM_SUBCORES = 16
NW = NUM_CORES * NUM_SUBCORES
NB = 8

_mesh = plsc.VectorSubcoreMesh(core_axis_name="c", subcore_axis_name="s")


@functools.lru_cache(maxsize=None)
def _make_gather(batch, hist):
    assert batch % (NW * NB * 2) == 0 and hist <= 128
    hp = ((hist + 7) // 8) * 8
    per_w = batch // NW
    groups = per_w // NB
    gr = NB * hp

    @functools.partial(
        pl.kernel,
        out_type=jax.ShapeDtypeStruct((batch * hp, 2 * HIDDEN), jnp.float32),
        mesh=_mesh,
        scratch_types=[
            pltpu.VMEM((2, NB, hist), jnp.int32),
            pltpu.VMEM((2, gr, 2 * HIDDEN), jnp.float32),
            pltpu.SemaphoreType.DMA,
            pltpu.SemaphoreType.DMA,
            pltpu.SemaphoreType.DMA,
            pltpu.SemaphoreType.DMA,
        ],
        compiler_params=pltpu.CompilerParams(use_tc_tiling_on_sc=False),
    )
    def gather_kernel(table_hbm, idx_hbm, out_hbm, idx_v, rows_v,
                      gsem0, gsem1, osem0, osem1):
        wid = lax.axis_index("s") * NUM_CORES + lax.axis_index("c")
        base_b = wid * per_w
        base_r = base_b * hp
        gsems = (gsem0, gsem1)
        osems = (osem0, osem1)

        def fire(b, g):
            pltpu.sync_copy(idx_hbm.at[pl.ds(base_b + g * NB, NB)], idx_v.at[b])
            for j in range(NB):
                pltpu.async_copy(table_hbm.at[idx_v.at[b].at[j]],
                                 rows_v.at[b].at[pl.ds(j * hp, hist)],
                                 gsems[b])

        def drain_gathers(b):
            for j in range(NB):
                pltpu.make_async_copy(table_hbm.at[idx_v.at[b].at[j]],
                                      rows_v.at[b].at[pl.ds(j * hp, hist)],
                                      gsems[b]).wait()

        def out_slice(g):
            return out_hbm.at[pl.ds(base_r + g * gr, gr), pl.ds(0, HIDDEN)]

        def fire_out(b, g):
            pltpu.async_copy(rows_v.at[b].at[:, pl.ds(0, HIDDEN)],
                             out_slice(g), osems[b])

        def wait_out(b, g):
            pltpu.make_async_copy(rows_v.at[b].at[:, pl.ds(0, HIDDEN)],
                                  out_slice(g), osems[b]).wait()

        fire(0, 0)

        @pl.loop(0, groups, step=2)
        def _grp(g0):
            for b in (0, 1):
                gg = g0 + b
                nb = 1 - b
                gf = gg + 1

                @pl.when(gf < groups)
                def _():
                    @pl.when(gg >= 1)
                    def _():
                        wait_out(nb, gg - 1)
                    fire(nb, gf)

                drain_gathers(b)
                fire_out(b, gg)

        wait_out(0, groups - 2)
        wait_out(1, groups - 1)

    return gather_kernel


def kernel(inputs, weight):
    batch, hist = inputs.shape
    hp = ((hist + 7) // 8) * 8
    idx = inputs.astype(jnp.int32)
    w128 = jnp.pad(weight, ((0, 0), (0, 2 * HIDDEN - weight.shape[1])))
    out = _make_gather(batch, hist)(w128, idx)
    return out.reshape(batch, hp, 2 * HIDDEN)[:, :hist, :HIDDEN]


# speedup vs baseline: 1.1742x; 1.1112x over previous
import functools

import jax
import jax.numpy as jnp
from jax import lax
from jax.experimental import pallas as pl
from jax.experimental.pallas import tpu as pltpu
from jax.experimental.pallas import tpu_sc as plsc

HIDDEN = 64
NUM_CORES = 2
NUM_SUBCORES = 16
NW = NUM_CORES * NUM_SUBCORES
NB = 16

_mesh = plsc.VectorSubcoreMesh(core_axis_name="c", subcore_axis_name="s")


@functools.lru_cache(maxsize=None)
def _make_gather(batch, hist):
    assert batch % (NW * NB * 2) == 0 and hist <= 128
    hp = ((hist + 7) // 8) * 8
    per_w = batch // NW
    groups = per_w // NB
    gr = NB * hp

    @functools.partial(
        pl.kernel,
        out_type=jax.ShapeDtypeStruct((batch * hp, 2 * HIDDEN), jnp.float32),
        mesh=_mesh,
        scratch_types=[
            pltpu.VMEM((2, NB, hist), jnp.int32),
            pltpu.VMEM((2, gr, HIDDEN), jnp.float32),
            pltpu.SemaphoreType.DMA,
            pltpu.SemaphoreType.DMA,
            pltpu.SemaphoreType.DMA,
            pltpu.SemaphoreType.DMA,
        ],
        compiler_params=pltpu.CompilerParams(use_tc_tiling_on_sc=False),
    )
    def gather_kernel(table_hbm, idx_hbm, out_hbm, idx_v, rows_v,
                      gsem0, gsem1, osem0, osem1):
        wid = lax.axis_index("s") * NUM_CORES + lax.axis_index("c")
        base_b = wid * per_w
        base_r = base_b * hp
        gsems = (gsem0, gsem1)
        osems = (osem0, osem1)

        def fire(b, g):
            pltpu.sync_copy(idx_hbm.at[pl.ds(base_b + g * NB, NB)], idx_v.at[b])
            for j in range(NB):
                pltpu.async_copy(table_hbm.at[idx_v.at[b].at[j]],
                                 rows_v.at[b].at[pl.ds(j * hp, hist)],
                                 gsems[b])

        def drain_gathers(b):
            for j in range(NB):
                pltpu.make_async_copy(table_hbm.at[idx_v.at[b].at[j]],
                                      rows_v.at[b].at[pl.ds(j * hp, hist)],
                                      gsems[b]).wait()

        def out_slice(g):
            return out_hbm.at[pl.ds(base_r + g * gr, gr), pl.ds(0, HIDDEN)]

        def fire_out(b, g):
            pltpu.async_copy(rows_v.at[b], out_slice(g), osems[b])

        def wait_out(b, g):
            pltpu.make_async_copy(rows_v.at[b], out_slice(g), osems[b]).wait()

        fire(0, 0)

        @pl.loop(0, groups, step=2)
        def _grp(g0):
            for b in (0, 1):
                gg = g0 + b
                nb = 1 - b
                gf = gg + 1

                @pl.when(gf < groups)
                def _():
                    @pl.when(gg >= 1)
                    def _():
                        wait_out(nb, gg - 1)
                    fire(nb, gf)

                drain_gathers(b)
                fire_out(b, gg)

        wait_out(0, groups - 2)
        wait_out(1, groups - 1)

    return gather_kernel


def kernel(inputs, weight):
    batch, hist = inputs.shape
    hp = ((hist + 7) // 8) * 8
    idx = inputs.astype(jnp.int32) * 2
    w128 = jnp.pad(weight, ((0, 0), (0, 2 * HIDDEN - weight.shape[1])))
    table = w128.reshape(2 * weight.shape[0], weight.shape[1])
    out = _make_gather(batch, hist)(table, idx)
    return out.reshape(batch, hp, 2 * HIDDEN)[:, :hist, :HIDDEN]
